# Initial kernel scaffold; baseline (speedup 1.0000x reference)
import jax
import jax.numpy as jnp
from jax.experimental import pallas as pl


def kernel(feat_lr, guide_hr, sx_raw, sy_raw, th_raw, sr_raw):
    raise NotImplementedError("write your pallas kernel here")


# speedup vs baseline: 704.4158x; 704.4158x over previous
import functools
import math

import numpy as np
import jax
import jax.numpy as jnp
from jax import lax
from jax.experimental import pallas as pl
from jax.experimental.pallas import tpu as pltpu
from jax.experimental.pallas import tpu_sc as plsc

Hl, Wl = 14, 14
SCALE = 16
Hh, Wh = Hl * SCALE, Wl * SCALE
C = 96
R_MAX = 4
K = 81
KP = 88
P = 256
NCELL = Hl * Wl
CT = 128
NW = 32
CHUNK = 128
NCHUNK = 5
BPW = NCHUNK * CHUNK
NROWS_PAD = NW * BPW

_rr = np.arange(Hl)[:, None, None, None]
_ss = np.arange(Wl)[None, :, None, None]
_wi = np.arange(9)[None, None, :, None]
_wj = np.arange(9)[None, None, None, :]
_idx81 = (np.clip(_rr + _wi - R_MAX, 0, Hl - 1) * Wl
          + np.clip(_ss + _wj - R_MAX, 0, Wl - 1)).reshape(NCELL, K)
_idxKP = np.zeros((NCELL, KP), np.int32)
_idxKP[:, :K] = _idx81
_idx_flat = np.zeros((NROWS_PAD,), np.int32)
_idx_flat[: NCELL * KP] = _idxKP.reshape(-1)
IDX3_NP = _idx_flat.reshape(NW, NCHUNK, CHUNK)


def _sc_gather(table):
    mesh = plsc.VectorSubcoreMesh(core_axis_name="c", subcore_axis_name="s")

    @functools.partial(
        pl.kernel,
        mesh=mesh,
        out_type=jax.ShapeDtypeStruct((NROWS_PAD, CT), jnp.float32),
        scratch_types=[
            pltpu.VMEM((NCHUNK, CHUNK), jnp.int32),
            pltpu.VMEM((BPW, CT), jnp.float32),
            pltpu.SemaphoreType.DMA,
        ],
    )
    def gather_kernel(table_hbm, idx_hbm, out_hbm, idx_v, rows_v, sem):
        wid = lax.axis_index("s") * 2 + lax.axis_index("c")
        pltpu.sync_copy(idx_hbm.at[wid], idx_v)
        for c in range(NCHUNK):
            pltpu.async_copy(
                table_hbm.at[idx_v.at[c]],
                rows_v.at[pl.ds(c * CHUNK, CHUNK)],
                sem,
            ).wait()
        pltpu.sync_copy(rows_v, out_hbm.at[pl.ds(wid * BPW, BPW)])

    return gather_kernel(table, jnp.asarray(IDX3_NP))


def _tc_body(w_ref, g_ref, o_ref):
    t = pl.program_id(0)
    r = t // Wl
    s = t - r * Wl

    col = lambda c: w_ref[:, c : c + 1]
    ycell, xcell = col(0), col(1)
    cos_t, sin_t = col(2), col(3)
    i2sx, i2sy, i2sr = col(4), col(5), col(6)
    g0, g1, g2 = col(7), col(8), col(9)

    ii = lax.broadcasted_iota(jnp.int32, (1, P), 1)
    iq = ii // SCALE
    jq = ii - iq * SCALE
    yv = (iq + r * SCALE).astype(jnp.float32)
    xv = (jq + s * SCALE).astype(jnp.float32)

    cy = (ycell + 0.5) * SCALE - 0.5
    cx = (xcell + 0.5) * SCALE - 0.5
    dy = yv - cy
    dx = xv - cx
    xp = dx * cos_t + dy * sin_t
    yp = dy * cos_t - dx * sin_t
    log_w = -(xp * xp * i2sx + yp * yp * i2sy)

    gh0 = g_ref[0, 0:1, :]
    gh1 = g_ref[0, 1:2, :]
    gh2 = g_ref[0, 2:3, :]
    d0 = gh0 - g0
    d1 = gh1 - g1
    d2 = gh2 - g2
    log_w = log_w - (d0 * d0 + d1 * d1 + d2 * d2) * i2sr

    se = lambda a, b: w_ref[30 + 9 * a + b : 31 + 9 * a + b, 10:11]
    ilo = iq < (SCALE // 2)
    jlo = jq < (SCALE // 2)
    rlo0 = jnp.where(jlo, se(0, 0), se(0, 1))
    rlo1 = jnp.where(jlo, se(1, 0), se(1, 1))
    rlo2 = jnp.where(jlo, se(2, 0), se(2, 1))
    rhi0 = jnp.where(jlo, se(0, 1), se(0, 2))
    rhi1 = jnp.where(jlo, se(1, 1), se(1, 2))
    rhi2 = jnp.where(jlo, se(2, 1), se(2, 2))
    v00 = jnp.where(ilo, rlo0, rlo1)
    v10 = jnp.where(ilo, rlo1, rlo2)
    v01 = jnp.where(ilo, rhi0, rhi1)
    v11 = jnp.where(ilo, rhi1, rhi2)
    i_f = iq.astype(jnp.float32)
    j_f = jq.astype(jnp.float32)
    ti = jnp.where(ilo, i_f + 8.5, i_f - 7.5) * (1.0 / SCALE)
    tj = jnp.where(jlo, j_f + 8.5, j_f - 7.5) * (1.0 / SCALE)
    se_hr = (1 - ti) * ((1 - tj) * v00 + tj * v01) + ti * ((1 - tj) * v10 + tj * v11)
    r_map = jnp.clip(jnp.ceil(2.0 * se_hr), 1.0, float(R_MAX))
    r2 = r_map * r_map

    kk = lax.broadcasted_iota(jnp.int32, (KP, 1), 0)
    wi = kk // 9
    wj = kk - wi * 9
    rad2 = ((wi - 4) * (wi - 4) + (wj - 4) * (wj - 4)).astype(jnp.float32)
    valid = (kk < K) & (rad2 <= r2)
    log_w = jnp.where(valid, log_w, -1e30)

    m = jnp.max(log_w, axis=0, keepdims=True)
    w = jnp.exp(log_w - m)
    den = jnp.sum(w, axis=0, keepdims=True)

    feat = w_ref[:, 11 : 11 + C]
    num = lax.dot_general(feat, w, (((0,), (0,)), ((), ())),
                          preferred_element_type=jnp.float32)
    o_ref[0] = num / jnp.maximum(den, 1e-8)


def _build_table(feat_lr, guide_hr, sx_raw, sy_raw, th_raw, sr_raw):
    sx = jnp.exp(sx_raw[0, 0])
    sy = jnp.exp(sy_raw[0, 0])
    th = math.pi * jnp.tanh(th_raw[0, 0])
    sr = jnp.exp(sr_raw[0, 0])
    sxm = jnp.maximum(sx, 1e-6)
    sym = jnp.maximum(sy, 1e-6)
    srm = jnp.maximum(sr, 1e-6)
    i2sx = 1.0 / (2.0 * sxm * sxm + 1e-8)
    i2sy = 1.0 / (2.0 * sym * sym + 1e-8)
    i2sr = 1.0 / (2.0 * srm * srm + 1e-8)
    se = jnp.maximum(sx, sy)

    gh = guide_hr[0]
    gl = 0.25 * (gh[:, 7::16, 7::16] + gh[:, 7::16, 8::16]
                 + gh[:, 8::16, 7::16] + gh[:, 8::16, 8::16])

    n = jnp.arange(NCELL, dtype=jnp.float32)
    ycell = jnp.floor(n * (1.0 / Wl))
    xcell = n - ycell * Wl
    flat = lambda a: a.reshape(NCELL)
    cols = jnp.stack(
        [ycell, xcell, flat(jnp.cos(th)), flat(jnp.sin(th)), flat(i2sx),
         flat(i2sy), flat(i2sr), flat(gl[0]), flat(gl[1]), flat(gl[2]),
         flat(se)], axis=1)
    feat_flat = jnp.transpose(feat_lr[0], (1, 2, 0)).reshape(NCELL, C)
    pad = jnp.zeros((NCELL, CT - 11 - C), jnp.float32)
    return jnp.concatenate([cols, feat_flat, pad], axis=1)


def kernel(feat_lr, guide_hr, sx_raw, sy_raw, th_raw, sr_raw):
    table = _build_table(feat_lr, guide_hr, sx_raw, sy_raw, th_raw, sr_raw)
    w_full = _sc_gather(table)

    guide_cm = (guide_hr[0].reshape(3, Hl, SCALE, Wl, SCALE)
                .transpose(1, 3, 0, 2, 4).reshape(NCELL, 3, P))

    out_cm = pl.pallas_call(
        _tc_body,
        grid=(NCELL,),
        in_specs=[
            pl.BlockSpec((KP, CT), lambda t: (t, 0)),
            pl.BlockSpec((1, 3, P), lambda t: (t, 0, 0)),
        ],
        out_specs=pl.BlockSpec((1, C, P), lambda t: (t, 0, 0)),
        out_shape=jax.ShapeDtypeStruct((NCELL, C, P), jnp.float32),
    )(w_full, guide_cm)

    return (out_cm.reshape(Hl, Wl, C, SCALE, SCALE)
            .transpose(2, 0, 3, 1, 4).reshape(1, C, Hh, Wh))


# speedup vs baseline: 708.1176x; 1.0053x over previous
import functools
import math

import numpy as np
import jax
import jax.numpy as jnp
from jax import lax
from jax.experimental import pallas as pl
from jax.experimental.pallas import tpu as pltpu
from jax.experimental.pallas import tpu_sc as plsc

Hl, Wl = 14, 14
SCALE = 16
Hh, Wh = Hl * SCALE, Wl * SCALE
C = 96
R_MAX = 4
K = 81
KP = 88
P = 256
NCELL = Hl * Wl
CT = 128
NW = 32
CHUNK = 128
NCHUNK = 5
BPW = NCHUNK * CHUNK
NROWS_PAD = NW * BPW

_rr = np.arange(Hl)[:, None, None, None]
_ss = np.arange(Wl)[None, :, None, None]
_wi = np.arange(9)[None, None, :, None]
_wj = np.arange(9)[None, None, None, :]
_idx81 = (np.clip(_rr + _wi - R_MAX, 0, Hl - 1) * Wl
          + np.clip(_ss + _wj - R_MAX, 0, Wl - 1)).reshape(NCELL, K)
_idxKP = np.zeros((NCELL, KP), np.int32)
_idxKP[:, :K] = _idx81
_idx_flat = np.zeros((NROWS_PAD,), np.int32)
_idx_flat[: NCELL * KP] = _idxKP.reshape(-1)
IDX3_NP = _idx_flat.reshape(NW, NCHUNK, CHUNK)


def _sc_gather(table):
    mesh = plsc.VectorSubcoreMesh(core_axis_name="c", subcore_axis_name="s")

    @functools.partial(
        pl.kernel,
        mesh=mesh,
        out_type=jax.ShapeDtypeStruct((NROWS_PAD, CT), jnp.float32),
        scratch_types=[
            pltpu.VMEM((NCHUNK, CHUNK), jnp.int32),
            pltpu.VMEM((BPW, CT), jnp.float32),
            pltpu.SemaphoreType.DMA,
        ],
    )
    def gather_kernel(table_hbm, idx_hbm, out_hbm, idx_v, rows_v, sem):
        wid = lax.axis_index("s") * 2 + lax.axis_index("c")
        pltpu.sync_copy(idx_hbm.at[wid], idx_v)
        copies = [
            pltpu.async_copy(
                table_hbm.at[idx_v.at[c]],
                rows_v.at[pl.ds(c * CHUNK, CHUNK)],
                sem,
            )
            for c in range(NCHUNK)
        ]
        for cp in copies:
            cp.wait()
        pltpu.sync_copy(rows_v, out_hbm.at[pl.ds(wid * BPW, BPW)])

    return gather_kernel(table, jnp.asarray(IDX3_NP))


def _tc_body(w_ref, g_ref, o_ref):
    t = pl.program_id(0)
    r = t // Wl
    s = t - r * Wl

    col = lambda c: w_ref[:, c : c + 1]
    ycell, xcell = col(0), col(1)
    cos_t, sin_t = col(2), col(3)
    i2sx, i2sy, i2sr = col(4), col(5), col(6)
    g0, g1, g2 = col(7), col(8), col(9)

    ii = lax.broadcasted_iota(jnp.int32, (1, P), 1)
    iq = ii // SCALE
    jq = ii - iq * SCALE
    yv = (iq + r * SCALE).astype(jnp.float32)
    xv = (jq + s * SCALE).astype(jnp.float32)

    cy = (ycell + 0.5) * SCALE - 0.5
    cx = (xcell + 0.5) * SCALE - 0.5
    dy = yv - cy
    dx = xv - cx
    xp = dx * cos_t + dy * sin_t
    yp = dy * cos_t - dx * sin_t
    log_w = -(xp * xp * i2sx + yp * yp * i2sy)

    gh0 = g_ref[0, 0:1, :]
    gh1 = g_ref[0, 1:2, :]
    gh2 = g_ref[0, 2:3, :]
    d0 = gh0 - g0
    d1 = gh1 - g1
    d2 = gh2 - g2
    log_w = log_w - (d0 * d0 + d1 * d1 + d2 * d2) * i2sr

    se = lambda a, b: w_ref[30 + 9 * a + b : 31 + 9 * a + b, 10:11]
    ilo = iq < (SCALE // 2)
    jlo = jq < (SCALE // 2)
    rlo0 = jnp.where(jlo, se(0, 0), se(0, 1))
    rlo1 = jnp.where(jlo, se(1, 0), se(1, 1))
    rlo2 = jnp.where(jlo, se(2, 0), se(2, 1))
    rhi0 = jnp.where(jlo, se(0, 1), se(0, 2))
    rhi1 = jnp.where(jlo, se(1, 1), se(1, 2))
    rhi2 = jnp.where(jlo, se(2, 1), se(2, 2))
    v00 = jnp.where(ilo, rlo0, rlo1)
    v10 = jnp.where(ilo, rlo1, rlo2)
    v01 = jnp.where(ilo, rhi0, rhi1)
    v11 = jnp.where(ilo, rhi1, rhi2)
    i_f = iq.astype(jnp.float32)
    j_f = jq.astype(jnp.float32)
    ti = jnp.where(ilo, i_f + 8.5, i_f - 7.5) * (1.0 / SCALE)
    tj = jnp.where(jlo, j_f + 8.5, j_f - 7.5) * (1.0 / SCALE)
    se_hr = (1 - ti) * ((1 - tj) * v00 + tj * v01) + ti * ((1 - tj) * v10 + tj * v11)
    r_map = jnp.clip(jnp.ceil(2.0 * se_hr), 1.0, float(R_MAX))
    r2 = r_map * r_map

    kk = lax.broadcasted_iota(jnp.int32, (KP, 1), 0)
    wi = kk // 9
    wj = kk - wi * 9
    rad2 = ((wi - 4) * (wi - 4) + (wj - 4) * (wj - 4)).astype(jnp.float32)
    valid = (kk < K) & (rad2 <= r2)
    log_w = jnp.where(valid, log_w, -1e30)

    m = jnp.max(log_w, axis=0, keepdims=True)
    w = jnp.exp(log_w - m)
    den = jnp.sum(w, axis=0, keepdims=True)

    feat = w_ref[:, 11 : 11 + C]
    num = lax.dot_general(feat, w, (((0,), (0,)), ((), ())),
                          preferred_element_type=jnp.float32)
    o_ref[0] = num / jnp.maximum(den, 1e-8)


def _build_table(feat_lr, guide_hr, sx_raw, sy_raw, th_raw, sr_raw):
    sx = jnp.exp(sx_raw[0, 0])
    sy = jnp.exp(sy_raw[0, 0])
    th = math.pi * jnp.tanh(th_raw[0, 0])
    sr = jnp.exp(sr_raw[0, 0])
    sxm = jnp.maximum(sx, 1e-6)
    sym = jnp.maximum(sy, 1e-6)
    srm = jnp.maximum(sr, 1e-6)
    i2sx = 1.0 / (2.0 * sxm * sxm + 1e-8)
    i2sy = 1.0 / (2.0 * sym * sym + 1e-8)
    i2sr = 1.0 / (2.0 * srm * srm + 1e-8)
    se = jnp.maximum(sx, sy)

    gh = guide_hr[0]
    gl = 0.25 * (gh[:, 7::16, 7::16] + gh[:, 7::16, 8::16]
                 + gh[:, 8::16, 7::16] + gh[:, 8::16, 8::16])

    n = jnp.arange(NCELL, dtype=jnp.float32)
    ycell = jnp.floor(n * (1.0 / Wl))
    xcell = n - ycell * Wl
    flat = lambda a: a.reshape(NCELL)
    cols = jnp.stack(
        [ycell, xcell, flat(jnp.cos(th)), flat(jnp.sin(th)), flat(i2sx),
         flat(i2sy), flat(i2sr), flat(gl[0]), flat(gl[1]), flat(gl[2]),
         flat(se)], axis=1)
    feat_flat = jnp.transpose(feat_lr[0], (1, 2, 0)).reshape(NCELL, C)
    pad = jnp.zeros((NCELL, CT - 11 - C), jnp.float32)
    return jnp.concatenate([cols, feat_flat, pad], axis=1)


def kernel(feat_lr, guide_hr, sx_raw, sy_raw, th_raw, sr_raw):
    table = _build_table(feat_lr, guide_hr, sx_raw, sy_raw, th_raw, sr_raw)
    w_full = _sc_gather(table)

    guide_cm = (guide_hr[0].reshape(3, Hl, SCALE, Wl, SCALE)
                .transpose(1, 3, 0, 2, 4).reshape(NCELL, 3, P))

    out_cm = pl.pallas_call(
        _tc_body,
        grid=(NCELL,),
        in_specs=[
            pl.BlockSpec((KP, CT), lambda t: (t, 0)),
            pl.BlockSpec((1, 3, P), lambda t: (t, 0, 0)),
        ],
        out_specs=pl.BlockSpec((1, C, P), lambda t: (t, 0, 0)),
        out_shape=jax.ShapeDtypeStruct((NCELL, C, P), jnp.float32),
    )(w_full, guide_cm)

    return (out_cm.reshape(Hl, Wl, C, SCALE, SCALE)
            .transpose(2, 0, 3, 1, 4).reshape(1, C, Hh, Wh))


# speedup vs baseline: 1213.8398x; 1.7142x over previous
import math

import numpy as np
import jax
import jax.numpy as jnp
from jax import lax
from jax.experimental import pallas as pl

Hl, Wl = 14, 14
SCALE = 16
Hh, Wh = Hl * SCALE, Wl * SCALE
C = 96
R_MAX = 4
K = 81
KP = 88
P = 256
NCELL = Hl * Wl
NCP = 224
CT = 128
FCOL = 16


def _tc_body(tab_ref, g_ref, o_ref):
    t = pl.program_id(0)
    r = t // Wl
    s = t - r * Wl

    kk = lax.broadcasted_iota(jnp.int32, (KP, 1), 0)
    wi = kk // 9
    wj = kk - wi * 9
    yn = jnp.clip(r + wi - R_MAX, 0, Hl - 1)
    xn = jnp.clip(s + wj - R_MAX, 0, Wl - 1)
    n_k = yn * Wl + xn
    n_iota = lax.broadcasted_iota(jnp.int32, (1, NCP), 1)
    onehot = (n_iota == n_k).astype(jnp.float32)
    win = lax.dot_general(onehot, tab_ref[...], (((1,), (0,)), ((), ())),
                          preferred_element_type=jnp.float32)

    col = lambda c: win[:, c : c + 1]
    cos_t, sin_t = col(0), col(1)
    i2sx, i2sy, i2sr = col(2), col(3), col(4)
    g0, g1, g2 = col(5), col(6), col(7)

    ii = lax.broadcasted_iota(jnp.int32, (1, P), 1)
    iq = ii // SCALE
    jq = ii - iq * SCALE
    yv = (iq + r * SCALE).astype(jnp.float32)
    xv = (jq + s * SCALE).astype(jnp.float32)

    cy = (yn.astype(jnp.float32) + 0.5) * SCALE - 0.5
    cx = (xn.astype(jnp.float32) + 0.5) * SCALE - 0.5
    dy = yv - cy
    dx = xv - cx
    xp = dx * cos_t + dy * sin_t
    yp = dy * cos_t - dx * sin_t
    log_w = -(xp * xp * i2sx + yp * yp * i2sy)

    gh0 = g_ref[0, 0:1, :]
    gh1 = g_ref[0, 1:2, :]
    gh2 = g_ref[0, 2:3, :]
    d0 = gh0 - g0
    d1 = gh1 - g1
    d2 = gh2 - g2
    log_w = log_w - (d0 * d0 + d1 * d1 + d2 * d2) * i2sr

    se = lambda a, b: win[30 + 9 * a + b : 31 + 9 * a + b, 8:9]
    ilo = iq < (SCALE // 2)
    jlo = jq < (SCALE // 2)
    rlo0 = jnp.where(jlo, se(0, 0), se(0, 1))
    rlo1 = jnp.where(jlo, se(1, 0), se(1, 1))
    rlo2 = jnp.where(jlo, se(2, 0), se(2, 1))
    rhi0 = jnp.where(jlo, se(0, 1), se(0, 2))
    rhi1 = jnp.where(jlo, se(1, 1), se(1, 2))
    rhi2 = jnp.where(jlo, se(2, 1), se(2, 2))
    v00 = jnp.where(ilo, rlo0, rlo1)
    v10 = jnp.where(ilo, rlo1, rlo2)
    v01 = jnp.where(ilo, rhi0, rhi1)
    v11 = jnp.where(ilo, rhi1, rhi2)
    i_f = iq.astype(jnp.float32)
    j_f = jq.astype(jnp.float32)
    ti = jnp.where(ilo, i_f + 8.5, i_f - 7.5) * (1.0 / SCALE)
    tj = jnp.where(jlo, j_f + 8.5, j_f - 7.5) * (1.0 / SCALE)
    se_hr = (1 - ti) * ((1 - tj) * v00 + tj * v01) + ti * ((1 - tj) * v10 + tj * v11)
    r_map = jnp.clip(jnp.ceil(2.0 * se_hr), 1.0, float(R_MAX))
    r2 = r_map * r_map

    rad2 = ((wi - 4) * (wi - 4) + (wj - 4) * (wj - 4)).astype(jnp.float32)
    valid = (kk < K) & (rad2 <= r2)
    log_w = jnp.where(valid, log_w, -1e30)

    m = jnp.max(log_w, axis=0, keepdims=True)
    w = jnp.exp(log_w - m)
    den = jnp.sum(w, axis=0, keepdims=True)

    feat = win[:, FCOL : FCOL + C]
    num = lax.dot_general(feat, w, (((0,), (0,)), ((), ())),
                          preferred_element_type=jnp.float32)
    o_ref[0] = num / jnp.maximum(den, 1e-8)


def _build_table(feat_lr, guide_hr, sx_raw, sy_raw, th_raw, sr_raw):
    sx = jnp.exp(sx_raw[0, 0])
    sy = jnp.exp(sy_raw[0, 0])
    th = math.pi * jnp.tanh(th_raw[0, 0])
    sr = jnp.exp(sr_raw[0, 0])
    sxm = jnp.maximum(sx, 1e-6)
    sym = jnp.maximum(sy, 1e-6)
    srm = jnp.maximum(sr, 1e-6)
    i2sx = 1.0 / (2.0 * sxm * sxm + 1e-8)
    i2sy = 1.0 / (2.0 * sym * sym + 1e-8)
    i2sr = 1.0 / (2.0 * srm * srm + 1e-8)
    se = jnp.maximum(sx, sy)

    gh = guide_hr[0]
    gl = 0.25 * (gh[:, 7::16, 7::16] + gh[:, 7::16, 8::16]
                 + gh[:, 8::16, 7::16] + gh[:, 8::16, 8::16])

    flat = lambda a: a.reshape(NCELL)
    cols = jnp.stack(
        [flat(jnp.cos(th)), flat(jnp.sin(th)), flat(i2sx), flat(i2sy),
         flat(i2sr), flat(gl[0]), flat(gl[1]), flat(gl[2]), flat(se)],
        axis=1)
    feat_flat = jnp.transpose(feat_lr[0], (1, 2, 0)).reshape(NCELL, C)
    table = jnp.concatenate(
        [cols, jnp.zeros((NCELL, FCOL - 9), jnp.float32), feat_flat,
         jnp.zeros((NCELL, CT - FCOL - C), jnp.float32)], axis=1)
    return jnp.concatenate(
        [table, jnp.zeros((NCP - NCELL, CT), jnp.float32)], axis=0)


def kernel(feat_lr, guide_hr, sx_raw, sy_raw, th_raw, sr_raw):
    table = _build_table(feat_lr, guide_hr, sx_raw, sy_raw, th_raw, sr_raw)

    guide_cm = (guide_hr[0].reshape(3, Hl, SCALE, Wl, SCALE)
                .transpose(1, 3, 0, 2, 4).reshape(NCELL, 3, P))

    out_cm = pl.pallas_call(
        _tc_body,
        grid=(NCELL,),
        in_specs=[
            pl.BlockSpec((NCP, CT), lambda t: (0, 0)),
            pl.BlockSpec((1, 3, P), lambda t: (t, 0, 0)),
        ],
        out_specs=pl.BlockSpec((1, C, P), lambda t: (t, 0, 0)),
        out_shape=jax.ShapeDtypeStruct((NCELL, C, P), jnp.float32),
    )(table, guide_cm)

    return (out_cm.reshape(Hl, Wl, C, SCALE, SCALE)
            .transpose(2, 0, 3, 1, 4).reshape(1, C, Hh, Wh))
